# Initial kernel scaffold; baseline (speedup 1.0000x reference)
import jax
import jax.numpy as jnp
from jax.experimental import pallas as pl


def kernel(pooled_map, winner_indices, height, width):
    raise NotImplementedError("write your pallas kernel here")


# speedup vs baseline: 86.4244x; 86.4244x over previous
import functools

import jax
import jax.numpy as jnp
from jax import lax
from jax.experimental import pallas as pl
from jax.experimental.pallas import tpu as pltpu
from jax.experimental.pallas import tpu_sc as plsc

_SPACING = 2
_NUM_WORKERS = 32
_ROWS_PER_CHUNK = 32


@functools.partial(jax.jit, static_argnums=(2, 3))
def _unpool_sc(pooled2d, winner2d, n_rows, pw):
    rows_per_worker = n_rows // _NUM_WORKERS
    n_chunks = rows_per_worker // _ROWS_PER_CHUNK
    R = _ROWS_PER_CHUNK
    W = _SPACING * pw

    mesh = plsc.VectorSubcoreMesh(core_axis_name="c", subcore_axis_name="s")

    @functools.partial(
        pl.kernel,
        mesh=mesh,
        out_type=jax.ShapeDtypeStruct((_SPACING * n_rows * W,), jnp.float32),
        compiler_params=pltpu.CompilerParams(
            use_tc_tiling_on_sc=False, needs_layout_passes=False
        ),
        scratch_types=[
            pltpu.VMEM((R * pw,), jnp.float32),
            pltpu.VMEM((R * pw,), jnp.int32),
            pltpu.VMEM((_SPACING * R * W,), jnp.float32),
        ],
    )
    def k(pooled_hbm, winner_hbm, out_hbm, p_buf, w_buf, o_buf):
        cid = lax.axis_index("c")
        sid = lax.axis_index("s")
        wid = sid * 2 + cid
        base = wid * rows_per_worker
        two_iota = lax.iota(jnp.int32, 16) * 2

        def chunk_body(ci, carry):
            row = base + ci * R
            pltpu.sync_copy(pooled_hbm.at[pl.ds(row * pw, R * pw)], p_buf)
            pltpu.sync_copy(winner_hbm.at[pl.ds(row * pw, R * pw)], w_buf)

            def row_body(r, c2):
                in_base = r * pw
                out_base = jnp.full((16,), 2 * r * W, dtype=jnp.int32) + two_iota
                for k16 in range(pw // 16):
                    p = p_buf[pl.ds(in_base + 16 * k16, 16)]
                    w = w_buf[pl.ds(in_base + 16 * k16, 16)]
                    idx_e = out_base + (32 * k16)
                    idx_o = idx_e + 1
                    idx_e2 = idx_e + W
                    idx_o2 = idx_e2 + 1
                    zero = jnp.zeros((16,), jnp.float32)
                    v0 = jnp.where(w == 0, p, zero)
                    v1 = jnp.where(w == 1, p, zero)
                    v2 = jnp.where(w == 2, p, zero)
                    v3 = jnp.where(w == 3, p, zero)
                    plsc.store_scatter(o_buf, [idx_e], v0)
                    plsc.store_scatter(o_buf, [idx_o], v1)
                    plsc.store_scatter(o_buf, [idx_e2], v2)
                    plsc.store_scatter(o_buf, [idx_o2], v3)
                return c2

            lax.fori_loop(0, R, row_body, 0)
            pltpu.sync_copy(
                o_buf, out_hbm.at[pl.ds(_SPACING * row * W, _SPACING * R * W)]
            )
            return carry

        lax.fori_loop(0, n_chunks, chunk_body, 0)

    return k(pooled2d, winner2d)


def kernel(pooled_map, winner_indices, height, width):
    B, C, PH, PW = pooled_map.shape
    n_rows = B * C * PH
    p2 = pooled_map.reshape(n_rows * PW)
    w2 = winner_indices.reshape(n_rows * PW)
    out = _unpool_sc(p2, w2, n_rows, PW)
    return out.reshape(B, C, PH * _SPACING, PW * _SPACING)


# speedup vs baseline: 116.9110x; 1.3528x over previous
import functools

import jax
import jax.numpy as jnp
from jax import lax
from jax.experimental import pallas as pl
from jax.experimental.pallas import tpu as pltpu
from jax.experimental.pallas import tpu_sc as plsc

_SPACING = 2
_NUM_WORKERS = 32
_ROWS_PER_CHUNK = 32


@functools.partial(jax.jit, static_argnums=(2, 3))
def _unpool_sc(pooled2d, winner2d, n_rows, pw):
    rows_per_worker = n_rows // _NUM_WORKERS
    n_chunks = rows_per_worker // _ROWS_PER_CHUNK
    R = _ROWS_PER_CHUNK
    W = _SPACING * pw
    in_sz = R * pw
    out_sz = _SPACING * R * W

    mesh = plsc.VectorSubcoreMesh(core_axis_name="c", subcore_axis_name="s")

    @functools.partial(
        pl.kernel,
        mesh=mesh,
        out_type=jax.ShapeDtypeStruct((_SPACING * n_rows * W,), jnp.float32),
        compiler_params=pltpu.CompilerParams(
            use_tc_tiling_on_sc=False, needs_layout_passes=False
        ),
        scratch_types=[
            pltpu.VMEM((in_sz,), jnp.float32),
            pltpu.VMEM((in_sz,), jnp.float32),
            pltpu.VMEM((in_sz,), jnp.int32),
            pltpu.VMEM((in_sz,), jnp.int32),
            pltpu.VMEM((out_sz,), jnp.float32),
            pltpu.VMEM((out_sz,), jnp.float32),
            pltpu.SemaphoreType.DMA,
            pltpu.SemaphoreType.DMA,
            pltpu.SemaphoreType.DMA,
            pltpu.SemaphoreType.DMA,
            pltpu.SemaphoreType.DMA,
            pltpu.SemaphoreType.DMA,
        ],
    )
    def k(pooled_hbm, winner_hbm, out_hbm,
          p0, p1, w0, w1, o0, o1, sp0, sp1, sw0, sw1, so0, so1):
        cid = lax.axis_index("c")
        sid = lax.axis_index("s")
        wid = sid * 2 + cid
        base = wid * rows_per_worker
        two_iota = lax.iota(jnp.int32, 16) * 2
        pbufs, wbufs, obufs = (p0, p1), (w0, w1), (o0, o1)
        sps, sws, sos = (sp0, sp1), (sw0, sw1), (so0, so1)

        def in_copies(c, b):
            off = (base + c * R) * pw
            return (
                pltpu.make_async_copy(
                    pooled_hbm.at[pl.ds(off, in_sz)], pbufs[b], sps[b]),
                pltpu.make_async_copy(
                    winner_hbm.at[pl.ds(off, in_sz)], wbufs[b], sws[b]),
            )

        def out_copy(c, b):
            off = _SPACING * (base + c * R) * W
            return pltpu.make_async_copy(
                obufs[b], out_hbm.at[pl.ds(off, out_sz)], sos[b])

        for cp in in_copies(0, 0):
            cp.start()

        def compute_chunk(b):
            p_buf, w_buf, o_buf = pbufs[b], wbufs[b], obufs[b]

            def row_body(r, c2):
                in_base = r * pw
                out_base = jnp.full((16,), 2 * r * W, dtype=jnp.int32) + two_iota
                for k16 in range(pw // 16):
                    p = p_buf[pl.ds(in_base + 16 * k16, 16)]
                    w = w_buf[pl.ds(in_base + 16 * k16, 16)]
                    idx_e = out_base + (32 * k16)
                    idx_o = idx_e + 1
                    idx_e2 = idx_e + W
                    idx_o2 = idx_e2 + 1
                    zero = jnp.zeros((16,), jnp.float32)
                    v0 = jnp.where(w == 0, p, zero)
                    v1 = jnp.where(w == 1, p, zero)
                    v2 = jnp.where(w == 2, p, zero)
                    v3 = jnp.where(w == 3, p, zero)
                    plsc.store_scatter(o_buf, [idx_e], v0)
                    plsc.store_scatter(o_buf, [idx_o], v1)
                    plsc.store_scatter(o_buf, [idx_e2], v2)
                    plsc.store_scatter(o_buf, [idx_o2], v3)
                return c2

            lax.fori_loop(0, R, row_body, 0)

        def outer(g, carry):
            for b in range(2):
                c = 2 * g + b
                nb = 1 - b

                @pl.when(c + 1 < n_chunks)
                def _():
                    for cp in in_copies(c + 1, nb):
                        cp.start()

                for cp in in_copies(c, b):
                    cp.wait()

                @pl.when(c >= 2)
                def _():
                    out_copy(c - 2, b).wait()

                compute_chunk(b)
                out_copy(c, b).start()
            return carry

        lax.fori_loop(0, n_chunks // 2, outer, 0)
        for b in range(2):
            out_copy(n_chunks - 2 + b, b).wait()

    return k(pooled2d, winner2d)


def kernel(pooled_map, winner_indices, height, width):
    B, C, PH, PW = pooled_map.shape
    n_rows = B * C * PH
    p2 = pooled_map.reshape(n_rows * PW)
    w2 = winner_indices.reshape(n_rows * PW)
    out = _unpool_sc(p2, w2, n_rows, PW)
    return out.reshape(B, C, PH * _SPACING, PW * _SPACING)


# speedup vs baseline: 339.6850x; 2.9055x over previous
import functools

import jax
import jax.numpy as jnp
from jax import lax
from jax.experimental import pallas as pl
from jax.experimental.pallas import tpu as pltpu
from jax.experimental.pallas import tpu_sc as plsc

_SPACING = 2
_NUM_WORKERS = 32
_ROWS_PER_CHUNK = 32


@functools.partial(jax.jit, static_argnums=(2, 3))
def _unpool_sc(pooled2d, winner2d, n_rows, pw):
    rows_per_worker = n_rows // _NUM_WORKERS
    n_chunks = rows_per_worker // _ROWS_PER_CHUNK
    R = _ROWS_PER_CHUNK
    W = _SPACING * pw

    mesh = plsc.VectorSubcoreMesh(core_axis_name="c", subcore_axis_name="s")

    @functools.partial(
        pl.kernel,
        mesh=mesh,
        out_type=jax.ShapeDtypeStruct((_SPACING * n_rows, W), jnp.float32),
        compiler_params=pltpu.CompilerParams(
            use_tc_tiling_on_sc=True, needs_layout_passes=False
        ),
        scratch_types=[
            pltpu.VMEM((R, pw), jnp.float32),
            pltpu.VMEM((R, pw), jnp.float32),
            pltpu.VMEM((R, pw), jnp.int32),
            pltpu.VMEM((R, pw), jnp.int32),
            pltpu.VMEM((_SPACING * R, W), jnp.float32),
            pltpu.VMEM((_SPACING * R, W), jnp.float32),
            pltpu.SemaphoreType.DMA,
            pltpu.SemaphoreType.DMA,
            pltpu.SemaphoreType.DMA,
            pltpu.SemaphoreType.DMA,
            pltpu.SemaphoreType.DMA,
            pltpu.SemaphoreType.DMA,
        ],
    )
    def k(pooled_hbm, winner_hbm, out_hbm,
          p0, p1, w0, w1, o0, o1, sp0, sp1, sw0, sw1, so0, so1):
        cid = lax.axis_index("c")
        sid = lax.axis_index("s")
        wid = sid * 2 + cid
        base = wid * rows_per_worker
        two_iota = lax.iota(jnp.int32, 16) * 2
        pbufs, wbufs, obufs = (p0, p1), (w0, w1), (o0, o1)
        sps, sws, sos = (sp0, sp1), (sw0, sw1), (so0, so1)

        def in_copies(c, b):
            row = base + c * R
            return (
                pltpu.make_async_copy(
                    pooled_hbm.at[pl.ds(row, R)], pbufs[b], sps[b]),
                pltpu.make_async_copy(
                    winner_hbm.at[pl.ds(row, R)], wbufs[b], sws[b]),
            )

        def out_copy(c, b):
            row = _SPACING * (base + c * R)
            return pltpu.make_async_copy(
                obufs[b], out_hbm.at[pl.ds(row, _SPACING * R)], sos[b])

        for cp in in_copies(0, 0):
            cp.start()

        def compute_chunk(b):
            p_buf, w_buf, o_buf = pbufs[b], wbufs[b], obufs[b]

            def row_body(r, c2):
                row0 = jnp.full((16,), 2 * r, dtype=jnp.int32)
                row1 = row0 + 1
                for k16 in range(pw // 16):
                    p = p_buf[r, pl.ds(16 * k16, 16)]
                    w = w_buf[r, pl.ds(16 * k16, 16)]
                    col_e = two_iota + (32 * k16)
                    col_o = col_e + 1
                    zero = jnp.zeros((16,), jnp.float32)
                    v0 = jnp.where(w == 0, p, zero)
                    v1 = jnp.where(w == 1, p, zero)
                    v2 = jnp.where(w == 2, p, zero)
                    v3 = jnp.where(w == 3, p, zero)
                    plsc.store_scatter(o_buf, [row0, col_e], v0)
                    plsc.store_scatter(o_buf, [row0, col_o], v1)
                    plsc.store_scatter(o_buf, [row1, col_e], v2)
                    plsc.store_scatter(o_buf, [row1, col_o], v3)
                return c2

            lax.fori_loop(0, R, row_body, 0)

        def outer(g, carry):
            for b in range(2):
                c = 2 * g + b
                nb = 1 - b

                @pl.when(c + 1 < n_chunks)
                def _():
                    for cp in in_copies(c + 1, nb):
                        cp.start()

                for cp in in_copies(c, b):
                    cp.wait()

                @pl.when(c >= 2)
                def _():
                    out_copy(c - 2, b).wait()

                compute_chunk(b)
                out_copy(c, b).start()
            return carry

        lax.fori_loop(0, n_chunks // 2, outer, 0)
        for b in range(2):
            out_copy(n_chunks - 2 + b, b).wait()

    return k(pooled2d, winner2d)


def kernel(pooled_map, winner_indices, height, width):
    B, C, PH, PW = pooled_map.shape
    n_rows = B * C * PH
    p2 = pooled_map.reshape(n_rows, PW)
    w2 = winner_indices.reshape(n_rows, PW)
    out = _unpool_sc(p2, w2, n_rows, PW)
    return out.reshape(B, C, PH * _SPACING, PW * _SPACING)


# speedup vs baseline: 340.1428x; 1.0013x over previous
import functools

import jax
import jax.numpy as jnp
from jax import lax
from jax.experimental import pallas as pl
from jax.experimental.pallas import tpu as pltpu
from jax.experimental.pallas import tpu_sc as plsc

_SPACING = 2
_NUM_WORKERS = 32
_ROWS_PER_CHUNK = 32


@functools.partial(jax.jit, static_argnums=(2, 3))
def _unpool_sc(pooled2d, winner2d, n_rows, pw):
    rows_per_worker = n_rows // _NUM_WORKERS
    n_chunks = rows_per_worker // _ROWS_PER_CHUNK
    R = _ROWS_PER_CHUNK
    W = _SPACING * pw

    mesh = plsc.VectorSubcoreMesh(core_axis_name="c", subcore_axis_name="s")

    @functools.partial(
        pl.kernel,
        mesh=mesh,
        out_type=jax.ShapeDtypeStruct((_SPACING * n_rows, W), jnp.float32),
        compiler_params=pltpu.CompilerParams(
            use_tc_tiling_on_sc=True, needs_layout_passes=False
        ),
        scratch_types=[
            pltpu.VMEM((R, pw), jnp.float32),
            pltpu.VMEM((R, pw), jnp.float32),
            pltpu.VMEM((R, pw), jnp.int32),
            pltpu.VMEM((R, pw), jnp.int32),
            pltpu.VMEM((_SPACING * R, W), jnp.float32),
            pltpu.VMEM((_SPACING * R, W), jnp.float32),
            pltpu.SemaphoreType.DMA,
            pltpu.SemaphoreType.DMA,
            pltpu.SemaphoreType.DMA,
            pltpu.SemaphoreType.DMA,
            pltpu.SemaphoreType.DMA,
            pltpu.SemaphoreType.DMA,
        ],
    )
    def k(pooled_hbm, winner_hbm, out_hbm,
          p0, p1, w0, w1, o0, o1, sp0, sp1, sw0, sw1, so0, so1):
        cid = lax.axis_index("c")
        sid = lax.axis_index("s")
        wid = sid * 2 + cid
        base = wid * rows_per_worker
        two_iota = lax.iota(jnp.int32, 16) * 2
        pbufs, wbufs, obufs = (p0, p1), (w0, w1), (o0, o1)
        sps, sws, sos = (sp0, sp1), (sw0, sw1), (so0, so1)

        def in_copies(c, b):
            row = base + c * R
            return (
                pltpu.make_async_copy(
                    pooled_hbm.at[pl.ds(row, R)], pbufs[b], sps[b]),
                pltpu.make_async_copy(
                    winner_hbm.at[pl.ds(row, R)], wbufs[b], sws[b]),
            )

        def out_copy(c, b):
            row = _SPACING * (base + c * R)
            return pltpu.make_async_copy(
                obufs[b], out_hbm.at[pl.ds(row, _SPACING * R)], sos[b])

        for cp in in_copies(0, 0):
            cp.start()

        col_e_k = [two_iota + (32 * k) for k in range(pw // 16)]
        col_o_k = [c + 1 for c in col_e_k]
        zero = jnp.zeros((16,), jnp.float32)
        zero_i = jnp.zeros((16,), jnp.int32)

        def compute_chunk(b):
            p_buf, w_buf, o_buf = pbufs[b], wbufs[b], obufs[b]

            def row_body(r, rows):
                row0, row1 = rows
                for k16 in range(pw // 16):
                    p = p_buf[r, pl.ds(16 * k16, 16)]
                    w = w_buf[r, pl.ds(16 * k16, 16)]
                    v0 = jnp.where(w == 0, p, zero)
                    v1 = jnp.where(w == 1, p, zero)
                    v2 = jnp.where(w == 2, p, zero)
                    v3 = jnp.where(w == 3, p, zero)
                    plsc.store_scatter(o_buf, [row0, col_e_k[k16]], v0)
                    plsc.store_scatter(o_buf, [row0, col_o_k[k16]], v1)
                    plsc.store_scatter(o_buf, [row1, col_e_k[k16]], v2)
                    plsc.store_scatter(o_buf, [row1, col_o_k[k16]], v3)
                return (row0 + 2, row1 + 2)

            lax.fori_loop(0, R, row_body, (zero_i, zero_i + 1))

        def outer(g, carry):
            for b in range(2):
                c = 2 * g + b
                nb = 1 - b

                @pl.when(c + 1 < n_chunks)
                def _():
                    for cp in in_copies(c + 1, nb):
                        cp.start()

                for cp in in_copies(c, b):
                    cp.wait()

                @pl.when(c >= 2)
                def _():
                    out_copy(c - 2, b).wait()

                compute_chunk(b)
                out_copy(c, b).start()
            return carry

        lax.fori_loop(0, n_chunks // 2, outer, 0)
        for b in range(2):
            out_copy(n_chunks - 2 + b, b).wait()

    return k(pooled2d, winner2d)


def kernel(pooled_map, winner_indices, height, width):
    B, C, PH, PW = pooled_map.shape
    n_rows = B * C * PH
    p2 = pooled_map.reshape(n_rows, PW)
    w2 = winner_indices.reshape(n_rows, PW)
    out = _unpool_sc(p2, w2, n_rows, PW)
    return out.reshape(B, C, PH * _SPACING, PW * _SPACING)


# speedup vs baseline: 398.5159x; 1.1716x over previous
import functools

import jax
import jax.numpy as jnp
from jax import lax
from jax.experimental import pallas as pl
from jax.experimental.pallas import tpu as pltpu
from jax.experimental.pallas import tpu_sc as plsc

_SPACING = 2
_NUM_WORKERS = 32
_ROWS_PER_CHUNK = 32


@functools.partial(jax.jit, static_argnums=(2, 3))
def _unpool_sc(pooled2d, winner2d, n_rows, pw):
    rows_per_worker = n_rows // _NUM_WORKERS
    n_chunks = rows_per_worker // _ROWS_PER_CHUNK
    R = _ROWS_PER_CHUNK
    W = _SPACING * pw

    mesh = plsc.VectorSubcoreMesh(core_axis_name="c", subcore_axis_name="s")

    @functools.partial(
        pl.kernel,
        mesh=mesh,
        out_type=jax.ShapeDtypeStruct((_SPACING * n_rows, W), jnp.float32),
        compiler_params=pltpu.CompilerParams(
            use_tc_tiling_on_sc=True, needs_layout_passes=False
        ),
        scratch_types=[
            pltpu.VMEM((R, pw), jnp.float32),
            pltpu.VMEM((R, pw), jnp.float32),
            pltpu.VMEM((R, pw), jnp.int32),
            pltpu.VMEM((R, pw), jnp.int32),
            pltpu.VMEM((_SPACING * R, W), jnp.float32),
            pltpu.VMEM((_SPACING * R, W), jnp.float32),
            pltpu.SemaphoreType.DMA,
            pltpu.SemaphoreType.DMA,
            pltpu.SemaphoreType.DMA,
            pltpu.SemaphoreType.DMA,
            pltpu.SemaphoreType.DMA,
            pltpu.SemaphoreType.DMA,
        ],
    )
    def k(pooled_hbm, winner_hbm, out_hbm,
          p0, p1, w0, w1, o0, o1, sp0, sp1, sw0, sw1, so0, so1):
        cid = lax.axis_index("c")
        sid = lax.axis_index("s")
        wid = sid * 2 + cid
        base = wid * rows_per_worker
        two_iota = lax.iota(jnp.int32, 16) * 2
        pbufs, wbufs, obufs = (p0, p1), (w0, w1), (o0, o1)
        sps, sws, sos = (sp0, sp1), (sw0, sw1), (so0, so1)

        def in_copies(c, b):
            row = base + c * R
            return (
                pltpu.make_async_copy(
                    pooled_hbm.at[pl.ds(row, R)], pbufs[b], sps[b]),
                pltpu.make_async_copy(
                    winner_hbm.at[pl.ds(row, R)], wbufs[b], sws[b]),
            )

        def out_copy(c, b):
            row = _SPACING * (base + c * R)
            return pltpu.make_async_copy(
                obufs[b], out_hbm.at[pl.ds(row, _SPACING * R)], sos[b])

        for cp in in_copies(0, 0):
            cp.start()

        col_e_k = [two_iota + (32 * k) for k in range(pw // 16)]
        col_o_k = [c + 1 for c in col_e_k]
        zero = jnp.zeros((16,), jnp.float32)

        def compute_chunk(b):
            p_buf, w_buf, o_buf = pbufs[b], wbufs[b], obufs[b]

            @plsc.parallel_loop(0, R, unroll=2)
            def row_body(r):
                row0 = jnp.full((16,), 2 * r, dtype=jnp.int32)
                row1 = row0 + 1
                for k16 in range(pw // 16):
                    p = p_buf[r, pl.ds(16 * k16, 16)]
                    w = w_buf[r, pl.ds(16 * k16, 16)]
                    v0 = jnp.where(w == 0, p, zero)
                    v1 = jnp.where(w == 1, p, zero)
                    v2 = jnp.where(w == 2, p, zero)
                    v3 = jnp.where(w == 3, p, zero)
                    plsc.store_scatter(o_buf, [row0, col_e_k[k16]], v0)
                    plsc.store_scatter(o_buf, [row0, col_o_k[k16]], v1)
                    plsc.store_scatter(o_buf, [row1, col_e_k[k16]], v2)
                    plsc.store_scatter(o_buf, [row1, col_o_k[k16]], v3)

        def outer(g, carry):
            for b in range(2):
                c = 2 * g + b
                nb = 1 - b

                @pl.when(c + 1 < n_chunks)
                def _():
                    for cp in in_copies(c + 1, nb):
                        cp.start()

                for cp in in_copies(c, b):
                    cp.wait()

                @pl.when(c >= 2)
                def _():
                    out_copy(c - 2, b).wait()

                compute_chunk(b)
                out_copy(c, b).start()
            return carry

        lax.fori_loop(0, n_chunks // 2, outer, 0)
        for b in range(2):
            out_copy(n_chunks - 2 + b, b).wait()

    return k(pooled2d, winner2d)


def kernel(pooled_map, winner_indices, height, width):
    B, C, PH, PW = pooled_map.shape
    n_rows = B * C * PH
    p2 = pooled_map.reshape(n_rows, PW)
    w2 = winner_indices.reshape(n_rows, PW)
    out = _unpool_sc(p2, w2, n_rows, PW)
    return out.reshape(B, C, PH * _SPACING, PW * _SPACING)
